# baseline (speedup 1.0000x reference)
import jax
import jax.numpy as jnp
from jax.experimental import pallas as pl

N = 1024
D = 64
H = 4
_DIMN = (((1,), (1,)), ((), ()))


def _gat_layer_inside(h, W, a_s, a_d, b):
    hw = jnp.dot(h, W, preferred_element_type=jnp.float32)
    acc = jnp.zeros((N, D), jnp.float32)
    for k in range(H):
        hh = hw[:, D * k:D * (k + 1)]
        ak_s = a_s[k:k + 1, :]
        ak_d = a_d[k:k + 1, :]
        as_row = jax.lax.dot_general(ak_s, hh, _DIMN,
                                     preferred_element_type=jnp.float32)
        ad_col = jax.lax.dot_general(hh, ak_d, _DIMN,
                                     preferred_element_type=jnp.float32)
        A = jnp.max(as_row)
        ea = jnp.exp(as_row - A)
        eb = jnp.exp(0.2 * (as_row - A))
        c = ad_col + A
        M = jnp.maximum(c, 0.2 * c)
        f1 = jnp.exp(c - M)
        f2 = jnp.exp(0.2 * c - M)
        cond = (as_row + ad_col) >= 0.0
        w = jnp.where(cond, f1 * ea, f2 * eb)
        num = jnp.dot(w, hh, preferred_element_type=jnp.float32)
        den = jnp.sum(w, axis=1, keepdims=True)
        acc = acc + num / den
    return acc * (1.0 / H) + b


def _model_kernel(x_ref, We_ref, be_ref, W1_ref, as1_ref, ad1_ref, b1_ref,
                  W2_ref, as2_ref, ad2_ref, b2_ref, Wo_ref, bo_ref, out_ref):
    x = x_ref[...]
    h = jnp.maximum(jnp.dot(x, We_ref[...],
                            preferred_element_type=jnp.float32)
                    + be_ref[...], 0.0)
    h = jnp.maximum(_gat_layer_inside(h, W1_ref[...], as1_ref[...],
                                      ad1_ref[...], b1_ref[...]), 0.0)
    h = jnp.maximum(_gat_layer_inside(h, W2_ref[...], as2_ref[...],
                                      ad2_ref[...], b2_ref[...]), 0.0)
    raw = jnp.dot(h, Wo_ref[...], preferred_element_type=jnp.float32) \
        + bo_ref[...]
    col = jax.lax.broadcasted_iota(jnp.int32, raw.shape, 1)
    clipped = jnp.clip(raw, -5.0, 5.0)
    res = jnp.where(col == 1, jnp.abs(clipped), clipped)
    res = jnp.where(col >= 2, raw, res)
    out_ref[...] = res


def kernel(x, We, be, W1, as1, ad1, b1, W2, as2, ad2, b2, Wa, ba, Wc, bc):
    Wo = jnp.zeros((D, 8), jnp.float32).at[:, 0:2].set(Wa).at[:, 2:3].set(Wc)
    bo = jnp.zeros((1, 8), jnp.float32).at[0, 0:2].set(ba).at[0, 2:3].set(bc)
    out = pl.pallas_call(
        _model_kernel,
        out_shape=jax.ShapeDtypeStruct((N, 8), jnp.float32),
    )(x, We, be.reshape(1, D), W1, as1, ad1, b1.reshape(1, D),
      W2, as2, ad2, b2.reshape(1, D), Wo, bo)
    logits = out[:, 0:2]
    values = out[:, 2]
    return (logits, values)


# speedup vs baseline: 1.3504x; 1.3504x over previous
import jax
import jax.numpy as jnp
from jax.experimental import pallas as pl

N = 1024
D = 64
H = 4
_DIMN = (((1,), (1,)), ((), ()))


def _gat_layer_inside(h, W, a_s, a_d, b, ones_col):
    hw = jnp.dot(h, W, preferred_element_type=jnp.float32)
    acc = None
    for k in range(H):
        hh = hw[:, D * k:D * (k + 1)]
        hh1 = jnp.concatenate([hh, ones_col], axis=1)
        ak_s = a_s[k:k + 1, :]
        ak_d = a_d[k:k + 1, :]
        as_row = jax.lax.dot_general(ak_s, hh, _DIMN,
                                     preferred_element_type=jnp.float32)
        ad_col = jnp.sum(hh * ak_d, axis=1, keepdims=True)
        A = jnp.max(as_row)
        ea = jnp.exp(as_row - A)
        eb = jnp.exp(0.2 * (as_row - A))
        c = ad_col + A
        M = jnp.maximum(c, 0.2 * c)
        f1 = jnp.exp(c - M)
        f2 = jnp.exp(0.2 * c - M)
        w = jnp.maximum(f1 * ea, f2 * eb)
        nd = jnp.dot(w, hh1, preferred_element_type=jnp.float32)
        contrib = nd[:, :D] / nd[:, D:D + 1]
        acc = contrib if acc is None else acc + contrib
    return acc * (1.0 / H) + b


def _model_kernel(x_ref, We_ref, be_ref, W1_ref, as1_ref, ad1_ref, b1_ref,
                  W2_ref, as2_ref, ad2_ref, b2_ref, Wa_ref, ba_ref,
                  Wc_ref, bc_ref, logits_ref, values_ref):
    x = x_ref[...]
    ones_col = jnp.ones((N, 1), jnp.float32)
    h = jnp.maximum(jnp.dot(x, We_ref[...],
                            preferred_element_type=jnp.float32)
                    + be_ref[...], 0.0)
    h = jnp.maximum(_gat_layer_inside(h, W1_ref[...], as1_ref[...],
                                      ad1_ref[...], b1_ref[...], ones_col), 0.0)
    h = jnp.maximum(_gat_layer_inside(h, W2_ref[...], as2_ref[...],
                                      ad2_ref[...], b2_ref[...], ones_col), 0.0)
    lg = jnp.dot(h, Wa_ref[...], preferred_element_type=jnp.float32) \
        + ba_ref[...]
    lg = jnp.clip(lg, -5.0, 5.0)
    col = jax.lax.broadcasted_iota(jnp.int32, lg.shape, 1)
    logits_ref[...] = jnp.where(col == 1, jnp.abs(lg), lg)
    values_ref[...] = jnp.dot(h, Wc_ref[...],
                              preferred_element_type=jnp.float32) + bc_ref[...]


def kernel(x, We, be, W1, as1, ad1, b1, W2, as2, ad2, b2, Wa, ba, Wc, bc):
    logits, values = pl.pallas_call(
        _model_kernel,
        out_shape=(jax.ShapeDtypeStruct((N, 2), jnp.float32),
                   jax.ShapeDtypeStruct((N, 1), jnp.float32)),
    )(x, We, be.reshape(1, D), W1, as1, ad1, b1.reshape(1, D),
      W2, as2, ad2, b2.reshape(1, D), Wa, ba.reshape(1, 2),
      Wc, bc.reshape(1, 1))
    return (logits, values.reshape(-1))


# speedup vs baseline: 1.3642x; 1.0101x over previous
import jax
import jax.numpy as jnp
from jax.experimental import pallas as pl

N = 1024
D = 64
H = 4
_DIMN = (((1,), (1,)), ((), ()))


def _gat_layer_inside(h, W, a_s, a_d, b, ones_col):
    hw = jnp.dot(h, W, preferred_element_type=jnp.float32)
    acc = None
    for k in range(H):
        hh = hw[:, D * k:D * (k + 1)]
        hh1 = jnp.concatenate([hh, ones_col], axis=1)
        ak_s = a_s[k:k + 1, :]
        ak_d = a_d[k:k + 1, :]
        as_row = jax.lax.dot_general(ak_s, hh, _DIMN,
                                     preferred_element_type=jnp.float32)
        ad_col = jnp.sum(hh * ak_d, axis=1, keepdims=True)
        A = jnp.max(as_row)
        ea = jnp.exp(as_row - A)
        eb = jnp.exp(0.2 * (as_row - A))
        c = ad_col + A
        M = jnp.maximum(c, 0.2 * c)
        g = jnp.exp(0.8 * (c - M))
        w = jnp.maximum(g * ea, eb)
        nd = jnp.dot(w, hh1, preferred_element_type=jnp.float32)
        contrib = nd[:, :D] / nd[:, D:D + 1]
        acc = contrib if acc is None else acc + contrib
    return acc * (1.0 / H) + b


def _model_kernel(x_ref, We_ref, be_ref, W1_ref, as1_ref, ad1_ref, b1_ref,
                  W2_ref, as2_ref, ad2_ref, b2_ref, Wa_ref, ba_ref,
                  Wc_ref, bc_ref, logits_ref, values_ref):
    x = x_ref[...]
    ones_col = jnp.ones((N, 1), jnp.float32)
    h = jnp.maximum(jnp.dot(x, We_ref[...],
                            preferred_element_type=jnp.float32)
                    + be_ref[...], 0.0)
    h = jnp.maximum(_gat_layer_inside(h, W1_ref[...], as1_ref[...],
                                      ad1_ref[...], b1_ref[...], ones_col), 0.0)
    h = jnp.maximum(_gat_layer_inside(h, W2_ref[...], as2_ref[...],
                                      ad2_ref[...], b2_ref[...], ones_col), 0.0)
    lg = jnp.dot(h, Wa_ref[...], preferred_element_type=jnp.float32) \
        + ba_ref[...]
    lg = jnp.clip(lg, -5.0, 5.0)
    col = jax.lax.broadcasted_iota(jnp.int32, lg.shape, 1)
    logits_ref[...] = jnp.where(col == 1, jnp.abs(lg), lg)
    values_ref[...] = jnp.dot(h, Wc_ref[...],
                              preferred_element_type=jnp.float32) + bc_ref[...]


def kernel(x, We, be, W1, as1, ad1, b1, W2, as2, ad2, b2, Wa, ba, Wc, bc):
    logits, values = pl.pallas_call(
        _model_kernel,
        out_shape=(jax.ShapeDtypeStruct((N, 2), jnp.float32),
                   jax.ShapeDtypeStruct((N, 1), jnp.float32)),
    )(x, We, be.reshape(1, D), W1, as1, ad1, b1.reshape(1, D),
      W2, as2, ad2, b2.reshape(1, D), Wa, ba.reshape(1, 2),
      Wc, bc.reshape(1, 1))
    return (logits, values.reshape(-1))


# speedup vs baseline: 1.4034x; 1.0288x over previous
import jax
import jax.numpy as jnp
from jax.experimental import pallas as pl

N = 1024
D = 64
H = 4
_DIMN = (((1,), (1,)), ((), ()))


def _gat_layer_inside(h, W, a_s, a_d, b, ones_col):
    hw = jnp.dot(h, W, preferred_element_type=jnp.float32)
    acc = None
    for k in range(H):
        hh = hw[:, D * k:D * (k + 1)]
        hh1 = jnp.concatenate([hh, ones_col], axis=1)
        ak_s = a_s[k:k + 1, :]
        ak_d = a_d[k:k + 1, :]
        as_row = jax.lax.dot_general(ak_s, hh, _DIMN,
                                     preferred_element_type=jnp.float32)
        ad_col = jnp.sum(hh * ak_d, axis=1, keepdims=True)
        A = jnp.max(as_row)
        ea = jnp.exp(as_row - A)
        eb = jnp.exp(0.2 * (as_row - A))
        R = A - jnp.min(as_row)
        c = ad_col + A
        g = jnp.exp(0.8 * jnp.minimum(c, jnp.minimum(R, 75.0)))
        w = jnp.maximum(g * ea, eb)
        nd = jnp.dot(w, hh1, preferred_element_type=jnp.float32)
        contrib = nd[:, :D] / nd[:, D:D + 1]
        acc = contrib if acc is None else acc + contrib
    return acc * (1.0 / H) + b


def _model_kernel(x_ref, We_ref, be_ref, W1_ref, as1_ref, ad1_ref, b1_ref,
                  W2_ref, as2_ref, ad2_ref, b2_ref, Wa_ref, ba_ref,
                  Wc_ref, bc_ref, logits_ref, values_ref):
    x = x_ref[...]
    ones_col = jnp.ones((N, 1), jnp.float32)
    h = jnp.maximum(jnp.dot(x, We_ref[...],
                            preferred_element_type=jnp.float32)
                    + be_ref[...], 0.0)
    h = jnp.maximum(_gat_layer_inside(h, W1_ref[...], as1_ref[...],
                                      ad1_ref[...], b1_ref[...], ones_col), 0.0)
    h = jnp.maximum(_gat_layer_inside(h, W2_ref[...], as2_ref[...],
                                      ad2_ref[...], b2_ref[...], ones_col), 0.0)
    lg = jnp.dot(h, Wa_ref[...], preferred_element_type=jnp.float32) \
        + ba_ref[...]
    lg = jnp.clip(lg, -5.0, 5.0)
    col = jax.lax.broadcasted_iota(jnp.int32, lg.shape, 1)
    logits_ref[...] = jnp.where(col == 1, jnp.abs(lg), lg)
    values_ref[...] = jnp.dot(h, Wc_ref[...],
                              preferred_element_type=jnp.float32) + bc_ref[...]


def kernel(x, We, be, W1, as1, ad1, b1, W2, as2, ad2, b2, Wa, ba, Wc, bc):
    logits, values = pl.pallas_call(
        _model_kernel,
        out_shape=(jax.ShapeDtypeStruct((N, 2), jnp.float32),
                   jax.ShapeDtypeStruct((N, 1), jnp.float32)),
    )(x, We, be.reshape(1, D), W1, as1, ad1, b1.reshape(1, D),
      W2, as2, ad2, b2.reshape(1, D), Wa, ba.reshape(1, 2),
      Wc, bc.reshape(1, 1))
    return (logits, values.reshape(-1))
